# baseline (speedup 1.0000x reference)
import functools

import jax
import jax.numpy as jnp
from jax import lax
from jax.experimental import pallas as pl
from jax.experimental.pallas import tpu as pltpu


def _fused_kernel(x_ref, w1_ref, b1_ref, w2_ref, b2_ref,
                  probs_ref, disp_ref, comb_ref, aux_ref,
                  acc_ref, pos_ref, val_ref, carry_ref, psum_ref,
                  *, ni, nj, bi, sb, E, CAP, S, K):
    i = pl.program_id(0)
    j = pl.program_id(1)

    @pl.when((i == 0) & (j == 0))
    def _():
        carry_ref[...] = jnp.zeros_like(carry_ref)
        psum_ref[...] = jnp.zeros_like(psum_ref)

    @pl.when(i > 0)
    def _():
        pos_sub = pos_ref[pl.ds(j * sb, sb)]
        val_sub = val_ref[pl.ds(j * sb, sb)]
        c_io = lax.broadcasted_iota(jnp.int32, (sb, E, CAP), 2)
        m = c_io == pos_sub
        disp_ref[...] = m.astype(jnp.float32).reshape(1, sb, E, CAP)
        comb_ref[...] = jnp.where(m, val_sub, 0.0).reshape(1, sb, E, CAP)

    @pl.when(i < ni)
    def _():
        h = jnp.maximum(
            jnp.dot(x_ref[...], w1_ref[...],
                    preferred_element_type=jnp.float32) + b1_ref[...], 0.0)
        contrib = jnp.dot(h, w2_ref[...], preferred_element_type=jnp.float32)

        @pl.when(j == 0)
        def _():
            acc_ref[...] = contrib

        @pl.when(j > 0)
        def _():
            acc_ref[...] = acc_ref[...] + contrib

        @pl.when(j == nj - 1)
        def _():
            lane = lax.broadcasted_iota(jnp.int32, (bi, 128), 1)
            valid = lane < E

            logit = jnp.where(valid, acc_ref[...] + b2_ref[...], -1e30)
            mx = jnp.max(logit, axis=1, keepdims=True)
            ex = jnp.where(valid, jnp.exp(logit - mx), 0.0)
            denom = jnp.sum(ex, axis=1, keepdims=True)
            probs = ex / denom
            probs_ref[...] = probs

            v0 = jnp.max(probs, axis=1, keepdims=True)
            idx0 = jnp.min(jnp.where(probs == v0, lane, 127),
                           axis=1, keepdims=True)
            masked = jnp.where(lane == idx0, -1.0, probs)
            v1 = jnp.max(masked, axis=1, keepdims=True)
            idx1 = jnp.min(jnp.where(masked == v1, lane, 127),
                           axis=1, keepdims=True)

            nrm = v0 + v1 + 1e-8
            p0 = v0 / nrm
            p1 = v1 / nrm

            oh0 = (lane == idx0).astype(jnp.float32)
            oh1 = (lane == idx1).astype(jnp.float32)
            rowcnt = oh0 + oh1

            r_io = lax.broadcasted_iota(jnp.int32, (bi, bi), 0)
            k_io = lax.broadcasted_iota(jnp.int32, (bi, bi), 1)
            tri = (k_io < r_io).astype(jnp.float32)
            excl = jnp.dot(tri, rowcnt, preferred_element_type=jnp.float32)
            excl = excl + carry_ref[...]

            pos0 = jnp.sum(excl * oh0, axis=1, keepdims=True).astype(jnp.int32)
            pos1 = jnp.sum(excl * oh1, axis=1, keepdims=True).astype(jnp.int32)

            carry_ref[...] = carry_ref[...] + jnp.sum(rowcnt, axis=0,
                                                      keepdims=True)
            psum_ref[...] = psum_ref[...] + jnp.sum(probs, axis=0,
                                                    keepdims=True)

            e_io8 = lax.broadcasted_iota(jnp.int32, (bi, E, 1), 1)
            hit0e = e_io8 == idx0.reshape(bi, 1, 1)
            hit1e = e_io8 == idx1.reshape(bi, 1, 1)
            q0 = jnp.where(pos0 < CAP, pos0, -1).reshape(bi, 1, 1)
            q1 = jnp.where(pos1 < CAP, pos1, -1).reshape(bi, 1, 1)
            pos_ref[...] = jnp.where(hit0e, q0, jnp.where(hit1e, q1, -1))
            val_ref[...] = jnp.where(hit0e, p0.reshape(bi, 1, 1),
                                     jnp.where(hit1e, p1.reshape(bi, 1, 1),
                                               0.0))

    @pl.when((i == ni) & (j == nj - 1))
    def _():
        usage = carry_ref[...] / float(S * K)
        pmean = psum_ref[...] / float(S)
        aux_ref[...] = jnp.sum(usage * pmean).reshape(1, 1) * float(E)


def _router(x, w1, b1r, w2p, b2p, *, bi, bj, E, CAP, K):
    s, hdim = x.shape
    ni, nj = s // bi, hdim // bj
    sb = bi // nj
    assert sb * nj == bi

    kfn = functools.partial(_fused_kernel, ni=ni, nj=nj, bi=bi, sb=sb,
                            E=E, CAP=CAP, S=s, K=K)
    return pl.pallas_call(
        kfn,
        grid=(ni + 1, nj),
        in_specs=[
            pl.BlockSpec((bi, hdim), lambda i, j: (jnp.minimum(i, ni - 1), 0)),
            pl.BlockSpec((hdim, bj),
                         lambda i, j: (0, jnp.where(i < ni, j, nj - 1))),
            pl.BlockSpec((1, bj),
                         lambda i, j: (0, jnp.where(i < ni, j, nj - 1))),
            pl.BlockSpec((bj, 128),
                         lambda i, j: (jnp.where(i < ni, j, nj - 1), 0)),
            pl.BlockSpec((1, 128), lambda i, j: (0, 0)),
        ],
        out_specs=[
            pl.BlockSpec((bi, 128), lambda i, j: (jnp.minimum(i, ni - 1), 0)),
            pl.BlockSpec((1, sb, E, CAP),
                         lambda i, j: (0, jnp.maximum(i - 1, 0) * nj + j,
                                       0, 0)),
            pl.BlockSpec((1, sb, E, CAP),
                         lambda i, j: (0, jnp.maximum(i - 1, 0) * nj + j,
                                       0, 0)),
            pl.BlockSpec((1, 1), lambda i, j: (0, 0)),
        ],
        out_shape=[
            jax.ShapeDtypeStruct((s, 128), jnp.float32),
            jax.ShapeDtypeStruct((1, s, E, CAP), jnp.float32),
            jax.ShapeDtypeStruct((1, s, E, CAP), jnp.float32),
            jax.ShapeDtypeStruct((1, 1), jnp.float32),
        ],
        scratch_shapes=[
            pltpu.VMEM((bi, 128), jnp.float32),
            pltpu.VMEM((bi, E, 1), jnp.int32),
            pltpu.VMEM((bi, E, 1), jnp.float32),
            pltpu.VMEM((1, 128), jnp.float32),
            pltpu.VMEM((1, 128), jnp.float32),
        ],
    )(x, w1, b1r, w2p, b2p)


def kernel(hidden_states, W1, b1, W2, b2):
    B, S, H = hidden_states.shape
    E = W2.shape[1]
    K = 2
    CF = 1.5
    CAP = int(B * S * CF * K / E)

    x = hidden_states.reshape(B * S, H)
    w2p = jnp.pad(W2, ((0, 0), (0, 128 - E)))
    b2p = jnp.pad(b2, (0, 128 - E)).reshape(1, 128)
    b1r = b1.reshape(1, H)

    probs_p, disp, comb, aux = _router(x, W1, b1r, w2p, b2p,
                                       bi=1024, bj=128, E=E, CAP=CAP, K=K)

    router_probs = probs_p[:, :E].reshape(B, S, E)
    return disp, comb, router_probs, aux[0, 0]


# speedup vs baseline: 1.6545x; 1.6545x over previous
import functools

import jax
import jax.numpy as jnp
from jax import lax
from jax.experimental import pallas as pl
from jax.experimental.pallas import tpu as pltpu


def _mm_kernel(x_ref, w1_ref, b1_ref, w2_ref, out_ref):
    j = pl.program_id(1)

    h = jnp.maximum(
        jnp.dot(x_ref[...], w1_ref[...], preferred_element_type=jnp.float32)
        + b1_ref[...], 0.0)
    contrib = jnp.dot(h, w2_ref[...], preferred_element_type=jnp.float32)

    @pl.when(j == 0)
    def _():
        out_ref[...] = contrib

    @pl.when(j > 0)
    def _():
        out_ref[...] = out_ref[...] + contrib


def _router_logits(x, w1, b1r, w2p, *, bi, bj):
    s, h = x.shape
    ni, nj = s // bi, h // bj
    return pl.pallas_call(
        _mm_kernel,
        grid=(ni, nj),
        in_specs=[
            pl.BlockSpec((bi, h), lambda i, j: (i, 0)),
            pl.BlockSpec((h, bj), lambda i, j: (0, j)),
            pl.BlockSpec((1, bj), lambda i, j: (0, j)),
            pl.BlockSpec((bj, 128), lambda i, j: (j, 0)),
        ],
        out_specs=pl.BlockSpec((bi, 128), lambda i, j: (i, 0)),
        out_shape=jax.ShapeDtypeStruct((s, 128), jnp.float32),
    )(x, w1, b1r, w2p)


def _route_kernel(logits_ref, b2_ref, probs_ref, disp_ref, comb_ref, aux_ref,
                  carry_ref, psum_ref, *, nb, bs, E, CAP, S, K):
    b = pl.program_id(0)

    @pl.when(b == 0)
    def _():
        carry_ref[...] = jnp.zeros_like(carry_ref)
        psum_ref[...] = jnp.zeros_like(psum_ref)

    lane = lax.broadcasted_iota(jnp.int32, (bs, 128), 1)
    valid = lane < E

    logit = jnp.where(valid, logits_ref[...] + b2_ref[...], -1e30)
    m = jnp.max(logit, axis=1, keepdims=True)
    ex = jnp.where(valid, jnp.exp(logit - m), 0.0)
    denom = jnp.sum(ex, axis=1, keepdims=True)
    probs = ex / denom
    probs_ref[...] = probs

    v0 = jnp.max(probs, axis=1, keepdims=True)
    idx0 = jnp.min(jnp.where(probs == v0, lane, 127), axis=1, keepdims=True)
    masked = jnp.where(lane == idx0, -1.0, probs)
    v1 = jnp.max(masked, axis=1, keepdims=True)
    idx1 = jnp.min(jnp.where(masked == v1, lane, 127), axis=1, keepdims=True)

    nrm = v0 + v1 + 1e-8
    p0 = v0 / nrm
    p1 = v1 / nrm

    oh0 = (lane == idx0).astype(jnp.float32)
    oh1 = (lane == idx1).astype(jnp.float32)
    rowcnt = oh0 + oh1

    r_io = lax.broadcasted_iota(jnp.int32, (bs, bs), 0)
    c_io = lax.broadcasted_iota(jnp.int32, (bs, bs), 1)
    tri = (c_io < r_io).astype(jnp.float32)
    excl = jnp.dot(tri, rowcnt, preferred_element_type=jnp.float32)
    excl = excl + carry_ref[...]

    pos0 = jnp.sum(excl * oh0, axis=1, keepdims=True).astype(jnp.int32)
    pos1 = jnp.sum(excl * oh1, axis=1, keepdims=True).astype(jnp.int32)

    carry_ref[...] = carry_ref[...] + jnp.sum(rowcnt, axis=0, keepdims=True)
    psum_ref[...] = psum_ref[...] + jnp.sum(probs, axis=0, keepdims=True)

    e_io8 = lax.broadcasted_iota(jnp.int32, (bs, E, 1), 1)
    hit0e = e_io8 == idx0.reshape(bs, 1, 1)
    hit1e = e_io8 == idx1.reshape(bs, 1, 1)
    q0 = jnp.where(pos0 < CAP, pos0, -1).reshape(bs, 1, 1)
    q1 = jnp.where(pos1 < CAP, pos1, -1).reshape(bs, 1, 1)
    pos_e = jnp.where(hit0e, q0, jnp.where(hit1e, q1, -1))
    val_e = jnp.where(hit0e, p0.reshape(bs, 1, 1),
                      jnp.where(hit1e, p1.reshape(bs, 1, 1), 0.0))

    c_io = lax.broadcasted_iota(jnp.int32, (bs, E, CAP), 2)
    m = c_io == pos_e
    disp_ref[...] = m.astype(jnp.float32).reshape(1, bs, E, CAP)
    comb_ref[...] = jnp.where(m, val_e, 0.0).reshape(1, bs, E, CAP)

    @pl.when(b == nb - 1)
    def _():
        usage = carry_ref[...] / float(S * K)
        pmean = psum_ref[...] / float(S)
        aux_ref[...] = jnp.sum(usage * pmean).reshape(1, 1) * float(E)


def _route(logits, b2p, *, bs, E, CAP, S, K):
    nb = S // bs
    return pl.pallas_call(
        functools.partial(_route_kernel, nb=nb, bs=bs, E=E, CAP=CAP, S=S, K=K),
        grid=(nb,),
        in_specs=[
            pl.BlockSpec((bs, 128), lambda b: (b, 0)),
            pl.BlockSpec((1, 128), lambda b: (0, 0)),
        ],
        out_specs=[
            pl.BlockSpec((bs, 128), lambda b: (b, 0)),
            pl.BlockSpec((1, bs, E, CAP), lambda b: (0, b, 0, 0)),
            pl.BlockSpec((1, bs, E, CAP), lambda b: (0, b, 0, 0)),
            pl.BlockSpec((1, 1), lambda b: (0, 0)),
        ],
        out_shape=[
            jax.ShapeDtypeStruct((S, 128), jnp.float32),
            jax.ShapeDtypeStruct((1, S, E, CAP), jnp.float32),
            jax.ShapeDtypeStruct((1, S, E, CAP), jnp.float32),
            jax.ShapeDtypeStruct((1, 1), jnp.float32),
        ],
        scratch_shapes=[
            pltpu.VMEM((1, 128), jnp.float32),
            pltpu.VMEM((1, 128), jnp.float32),
        ],
    )(logits, b2p)


def kernel(hidden_states, W1, b1, W2, b2):
    B, S, H = hidden_states.shape
    E = W2.shape[1]
    K = 2
    CF = 1.5
    CAP = int(B * S * CF * K / E)

    x = hidden_states.reshape(B * S, H)
    w2p = jnp.pad(W2, ((0, 0), (0, 128 - E)))
    b2p = jnp.pad(b2, (0, 128 - E)).reshape(1, 128)
    b1r = b1.reshape(1, H)

    logits = _router_logits(x, W1, b1r, w2p, bi=1024, bj=512)
    probs_p, disp, comb, aux = _route(logits, b2p, bs=256,
                                      E=E, CAP=CAP, S=B * S, K=K)

    router_probs = probs_p[:, :E].reshape(B, S, E)
    return disp, comb, router_probs, aux[0, 0]


# speedup vs baseline: 1.6589x; 1.0026x over previous
import functools

import jax
import jax.numpy as jnp
from jax import lax
from jax.experimental import pallas as pl
from jax.experimental.pallas import tpu as pltpu


def _mm_kernel(x_ref, w1_ref, b1_ref, w2_ref, out_ref):
    j = pl.program_id(0)

    h = jnp.maximum(
        jnp.dot(x_ref[...], w1_ref[...], preferred_element_type=jnp.float32)
        + b1_ref[...], 0.0)
    contrib = jnp.dot(h, w2_ref[...], preferred_element_type=jnp.float32)

    @pl.when(j == 0)
    def _():
        out_ref[...] = contrib

    @pl.when(j > 0)
    def _():
        out_ref[...] = out_ref[...] + contrib


def _router_logits(x, w1, b1r, w2p, *, bj):
    s, h = x.shape
    nj = h // bj
    return pl.pallas_call(
        _mm_kernel,
        grid=(nj,),
        in_specs=[
            pl.BlockSpec((s, h), lambda j: (0, 0)),
            pl.BlockSpec((h, bj), lambda j: (0, j)),
            pl.BlockSpec((1, bj), lambda j: (0, j)),
            pl.BlockSpec((bj, 128), lambda j: (j, 0)),
        ],
        out_specs=pl.BlockSpec((s, 128), lambda j: (0, 0)),
        out_shape=jax.ShapeDtypeStruct((s, 128), jnp.float32),
    )(x, w1, b1r, w2p)


def _route_kernel(logits_ref, b2_ref, probs_ref, disp_ref, comb_ref, aux_ref,
                  carry_ref, psum_ref, *, nb, bs, E, CAP, S, K):
    b = pl.program_id(0)

    @pl.when(b == 0)
    def _():
        carry_ref[...] = jnp.zeros_like(carry_ref)
        psum_ref[...] = jnp.zeros_like(psum_ref)

    lane = lax.broadcasted_iota(jnp.int32, (bs, 128), 1)
    valid = lane < E

    logit = jnp.where(valid, logits_ref[...] + b2_ref[...], -1e30)
    m = jnp.max(logit, axis=1, keepdims=True)
    ex = jnp.where(valid, jnp.exp(logit - m), 0.0)
    denom = jnp.sum(ex, axis=1, keepdims=True)
    probs = ex / denom
    probs_ref[...] = probs

    v0 = jnp.max(probs, axis=1, keepdims=True)
    idx0 = jnp.min(jnp.where(probs == v0, lane, 127), axis=1, keepdims=True)
    masked = jnp.where(lane == idx0, -1.0, probs)
    v1 = jnp.max(masked, axis=1, keepdims=True)
    idx1 = jnp.min(jnp.where(masked == v1, lane, 127), axis=1, keepdims=True)

    nrm = v0 + v1 + 1e-8
    p0 = v0 / nrm
    p1 = v1 / nrm

    oh0 = (lane == idx0).astype(jnp.float32)
    oh1 = (lane == idx1).astype(jnp.float32)
    rowcnt = oh0 + oh1

    r_io = lax.broadcasted_iota(jnp.int32, (bs, bs), 0)
    c_io = lax.broadcasted_iota(jnp.int32, (bs, bs), 1)
    tri = (c_io < r_io).astype(jnp.float32)
    excl = jnp.dot(tri, rowcnt, preferred_element_type=jnp.float32)
    excl = excl + carry_ref[...]

    pos0 = jnp.sum(excl * oh0, axis=1, keepdims=True).astype(jnp.int32)
    pos1 = jnp.sum(excl * oh1, axis=1, keepdims=True).astype(jnp.int32)

    carry_ref[...] = carry_ref[...] + jnp.sum(rowcnt, axis=0, keepdims=True)
    psum_ref[...] = psum_ref[...] + jnp.sum(probs, axis=0, keepdims=True)

    e_io8 = lax.broadcasted_iota(jnp.int32, (bs, E, 1), 1)
    hit0e = e_io8 == idx0.reshape(bs, 1, 1)
    hit1e = e_io8 == idx1.reshape(bs, 1, 1)
    q0 = jnp.where(pos0 < CAP, pos0, -1).reshape(bs, 1, 1)
    q1 = jnp.where(pos1 < CAP, pos1, -1).reshape(bs, 1, 1)
    pos_e = jnp.where(hit0e, q0, jnp.where(hit1e, q1, -1))
    val_e = jnp.where(hit0e, p0.reshape(bs, 1, 1),
                      jnp.where(hit1e, p1.reshape(bs, 1, 1), 0.0))

    c_io = lax.broadcasted_iota(jnp.int32, (bs, E, CAP), 2)
    m = c_io == pos_e
    disp_ref[...] = m.astype(jnp.float32).reshape(1, bs, E, CAP)
    comb_ref[...] = jnp.where(m, val_e, 0.0).reshape(1, bs, E, CAP)

    @pl.when(b == nb - 1)
    def _():
        usage = carry_ref[...] / float(S * K)
        pmean = psum_ref[...] / float(S)
        aux_ref[...] = jnp.sum(usage * pmean).reshape(1, 1) * float(E)


def _route(logits, b2p, *, bs, E, CAP, S, K):
    nb = S // bs
    return pl.pallas_call(
        functools.partial(_route_kernel, nb=nb, bs=bs, E=E, CAP=CAP, S=S, K=K),
        grid=(nb,),
        in_specs=[
            pl.BlockSpec((bs, 128), lambda b: (b, 0)),
            pl.BlockSpec((1, 128), lambda b: (0, 0)),
        ],
        out_specs=[
            pl.BlockSpec((bs, 128), lambda b: (b, 0)),
            pl.BlockSpec((1, bs, E, CAP), lambda b: (0, b, 0, 0)),
            pl.BlockSpec((1, bs, E, CAP), lambda b: (0, b, 0, 0)),
            pl.BlockSpec((1, 1), lambda b: (0, 0)),
        ],
        out_shape=[
            jax.ShapeDtypeStruct((S, 128), jnp.float32),
            jax.ShapeDtypeStruct((1, S, E, CAP), jnp.float32),
            jax.ShapeDtypeStruct((1, S, E, CAP), jnp.float32),
            jax.ShapeDtypeStruct((1, 1), jnp.float32),
        ],
        scratch_shapes=[
            pltpu.VMEM((1, 128), jnp.float32),
            pltpu.VMEM((1, 128), jnp.float32),
        ],
    )(logits, b2p)


def kernel(hidden_states, W1, b1, W2, b2):
    B, S, H = hidden_states.shape
    E = W2.shape[1]
    K = 2
    CF = 1.5
    CAP = int(B * S * CF * K / E)

    x = hidden_states.reshape(B * S, H)
    w2p = jnp.pad(W2, ((0, 0), (0, 128 - E)))
    b2p = jnp.pad(b2, (0, 128 - E)).reshape(1, 128)
    b1r = b1.reshape(1, H)

    logits = _router_logits(x, W1, b1r, w2p, bj=512)
    probs_p, disp, comb, aux = _route(logits, b2p, bs=256,
                                      E=E, CAP=CAP, S=B * S, K=K)

    router_probs = probs_p[:, :E].reshape(B, S, E)
    return disp, comb, router_probs, aux[0, 0]
